# Initial kernel scaffold; baseline (speedup 1.0000x reference)
import functools

import jax
import jax.numpy as jnp
from jax import lax
from jax.experimental import pallas as pl
from jax.experimental.pallas import tpu as pltpu
from jax.experimental.pallas import tpu_sc as plsc

N = 10000
DH = 16
NC = 2
NS = 16
NW = NC * NS
CHUNK = 128
CH_PER_TILE = 81
E_PAD = NW * CH_PER_TILE * CHUNK
ACC_ROWS = 10016


def _deg_body(dst3, zeros, ones_blk, out, dst_v, ones_v, acc):
    c = lax.axis_index("c")
    s = lax.axis_index("s")
    wid = c * NS + s
    pltpu.sync_copy(dst3.at[wid], dst_v)
    pltpu.sync_copy(ones_blk, ones_v)
    rps = ACC_ROWS // NS
    pltpu.sync_copy(zeros.at[pl.ds(s * rps, rps)], acc.at[pl.ds(s * rps, rps)])
    plsc.subcore_barrier()

    def chunk(j, carry):
        pltpu.sync_copy(ones_v, acc.at[dst_v.at[j]], add=True)
        return carry

    lax.fori_loop(0, CH_PER_TILE, chunk, 0)
    plsc.subcore_barrier()
    pltpu.sync_copy(acc.at[pl.ds(s * rps, rps)],
                    out.at[pl.ds(c * ACC_ROWS + s * rps, rps)])


def _agg_body(table, src3, dst3, zeros, out, src_v, dst_v, rows_v, acc):
    c = lax.axis_index("c")
    s = lax.axis_index("s")
    wid = c * NS + s
    pltpu.sync_copy(src3.at[wid], src_v)
    pltpu.sync_copy(dst3.at[wid], dst_v)
    rps = ACC_ROWS // NS
    pltpu.sync_copy(zeros.at[pl.ds(s * rps, rps)], acc.at[pl.ds(s * rps, rps)])
    plsc.subcore_barrier()

    def chunk(j, carry):
        pltpu.sync_copy(table.at[src_v.at[j]], rows_v)
        pltpu.sync_copy(rows_v, acc.at[dst_v.at[j]], add=True)
        return carry

    lax.fori_loop(0, CH_PER_TILE, chunk, 0)
    plsc.subcore_barrier()
    pltpu.sync_copy(acc.at[pl.ds(s * rps, rps)],
                    out.at[pl.ds(c * ACC_ROWS + s * rps, rps)])


_MESH = plsc.VectorSubcoreMesh(core_axis_name="c", subcore_axis_name="s")

_deg_call = pl.kernel(
    _deg_body,
    out_type=jax.ShapeDtypeStruct((NC * ACC_ROWS, DH), jnp.float32),
    mesh=_MESH,
    scratch_types=[
        pltpu.VMEM((CH_PER_TILE, CHUNK), jnp.int32),
        pltpu.VMEM((CHUNK, DH), jnp.float32),
        pltpu.VMEM_SHARED((ACC_ROWS, DH), jnp.float32),
    ],
)

_agg_call = pl.kernel(
    _agg_body,
    out_type=jax.ShapeDtypeStruct((NC * ACC_ROWS, DH), jnp.float32),
    mesh=_MESH,
    scratch_types=[
        pltpu.VMEM((CH_PER_TILE, CHUNK), jnp.int32),
        pltpu.VMEM((CH_PER_TILE, CHUNK), jnp.int32),
        pltpu.VMEM((CHUNK, DH), jnp.float32),
        pltpu.VMEM_SHARED((ACC_ROWS, DH), jnp.float32),
    ],
)


def _tc1_body(x_ref, w1_ref, degp_ref, h1p_ref, dinv_ref):
    deg = degp_ref[0:N, 0:1] + degp_ref[ACC_ROWS:ACC_ROWS + N, 0:1]
    dinv = jnp.where(deg > 0, lax.rsqrt(deg), 0.0)
    h = jnp.dot(x_ref[...], w1_ref[...], preferred_element_type=jnp.float32)
    h1p_ref[...] = h * dinv
    dinv_ref[...] = dinv


def _tc2_body(accp_ref, dinv_ref, b1_ref, g_ref):
    a = accp_ref[0:N, :] + accp_ref[ACC_ROWS:ACC_ROWS + N, :]
    dinv = dinv_ref[...]
    o = jnp.maximum(a * dinv + b1_ref[...], 0.0)
    g_ref[...] = o * dinv


def _tc3_body(accp_ref, dinv_ref, w2_ref, b2_ref, out_ref):
    a = (accp_ref[0:N, :] + accp_ref[ACC_ROWS:ACC_ROWS + N, :]) * dinv_ref[...]
    t = jnp.dot(a, w2_ref[...], preferred_element_type=jnp.float32) + b2_ref[...]
    m = jnp.max(t, axis=1, keepdims=True)
    out_ref[...] = (t - m) - jnp.log(
        jnp.sum(jnp.exp(t - m), axis=1, keepdims=True))


_tc1 = pl.pallas_call(
    _tc1_body,
    out_shape=[jax.ShapeDtypeStruct((N, DH), jnp.float32),
               jax.ShapeDtypeStruct((N, 1), jnp.float32)],
)

_tc2 = pl.pallas_call(
    _tc2_body,
    out_shape=jax.ShapeDtypeStruct((N, DH), jnp.float32),
)

_tc3 = pl.pallas_call(
    _tc3_body,
    out_shape=jax.ShapeDtypeStruct((N, 2), jnp.float32),
)


def kernel(x, edge_index, W1, b1, W2, b2):
    e = edge_index.astype(jnp.int32)
    loops = jnp.arange(N, dtype=jnp.int32)
    src = jnp.concatenate([e[0], loops])
    dst = jnp.concatenate([e[1], loops])
    pad = E_PAD - src.shape[0]
    src = jnp.concatenate([src, jnp.zeros((pad,), jnp.int32)])
    dst = jnp.concatenate([dst, jnp.full((pad,), N, jnp.int32)])
    src3 = src.reshape(NW, CH_PER_TILE, CHUNK)
    dst3 = dst.reshape(NW, CH_PER_TILE, CHUNK)

    zeros = jnp.zeros((ACC_ROWS, DH), jnp.float32)
    ones_blk = jnp.ones((CHUNK, DH), jnp.float32)

    degp = _deg_call(dst3, zeros, ones_blk)
    h1p, dinv = _tc1(x, W1, degp)
    acc1 = _agg_call(h1p, src3, dst3, zeros)
    g = _tc2(acc1, dinv, b1.reshape(1, DH))
    acc2 = _agg_call(g, src3, dst3, zeros)
    return _tc3(acc2, dinv, W2, b2.reshape(1, 2))


# speedup vs baseline: 34.0210x; 34.0210x over previous
import functools

import jax
import jax.numpy as jnp
from jax import lax
from jax.experimental import pallas as pl
from jax.experimental.pallas import tpu as pltpu
from jax.experimental.pallas import tpu_sc as plsc

N = 10000
DH = 16
NC = 2
NS = 16
NW = NC * NS
CHUNK = 128
CH_PER_TILE = 81
E_PAD = NW * CH_PER_TILE * CHUNK
ACC_ROWS = 10112


def _deg_body(dst3, zeros, ones_blk, out, dst_v, ones_v, acc):
    c = lax.axis_index("c")
    s = lax.axis_index("s")
    wid = c * NS + s
    pltpu.sync_copy(dst3.at[wid], dst_v)
    pltpu.sync_copy(ones_blk, ones_v)
    rps = ACC_ROWS // NS
    pltpu.sync_copy(zeros.at[pl.ds(s * rps, rps)], acc.at[pl.ds(s * rps, rps)])
    plsc.subcore_barrier()

    def chunk(j, carry):
        pltpu.sync_copy(ones_v, acc.at[dst_v.at[j]], add=True)
        return carry

    lax.fori_loop(0, CH_PER_TILE, chunk, 0)
    plsc.subcore_barrier()
    pltpu.sync_copy(acc.at[pl.ds(s * rps, rps)],
                    out.at[pl.ds(c * ACC_ROWS + s * rps, rps)])


def _agg_body(table, src3, dst3, zeros, out, src_v, dst_v, rows_v, acc):
    c = lax.axis_index("c")
    s = lax.axis_index("s")
    wid = c * NS + s
    pltpu.sync_copy(src3.at[wid], src_v)
    pltpu.sync_copy(dst3.at[wid], dst_v)
    rps = ACC_ROWS // NS
    pltpu.sync_copy(zeros.at[pl.ds(s * rps, rps)], acc.at[pl.ds(s * rps, rps)])
    plsc.subcore_barrier()

    def chunk(j, carry):
        pltpu.sync_copy(table.at[src_v.at[j]], rows_v)
        pltpu.sync_copy(rows_v, acc.at[dst_v.at[j]], add=True)
        return carry

    lax.fori_loop(0, CH_PER_TILE, chunk, 0)
    plsc.subcore_barrier()
    pltpu.sync_copy(acc.at[pl.ds(s * rps, rps)],
                    out.at[pl.ds(c * ACC_ROWS + s * rps, rps)])


_MESH = plsc.VectorSubcoreMesh(core_axis_name="c", subcore_axis_name="s")
_SC_PARAMS = pltpu.CompilerParams(use_tc_tiling_on_sc=False)

_deg_call = pl.kernel(
    _deg_body,
    out_type=jax.ShapeDtypeStruct((NC * ACC_ROWS, DH), jnp.float32),
    mesh=_MESH,
    compiler_params=_SC_PARAMS,
    scratch_types=[
        pltpu.VMEM((CH_PER_TILE, CHUNK), jnp.int32),
        pltpu.VMEM((CHUNK, DH), jnp.float32),
        pltpu.VMEM_SHARED((ACC_ROWS, DH), jnp.float32),
    ],
)

_agg_call = pl.kernel(
    _agg_body,
    out_type=jax.ShapeDtypeStruct((NC * ACC_ROWS, DH), jnp.float32),
    mesh=_MESH,
    compiler_params=_SC_PARAMS,
    scratch_types=[
        pltpu.VMEM((CH_PER_TILE, CHUNK), jnp.int32),
        pltpu.VMEM((CH_PER_TILE, CHUNK), jnp.int32),
        pltpu.VMEM((CHUNK, DH), jnp.float32),
        pltpu.VMEM_SHARED((ACC_ROWS, DH), jnp.float32),
    ],
)


def _tc1_body(x_ref, w1_ref, degp_ref, h1p_ref, dinv_ref):
    deg = degp_ref[0:N, 0:1] + degp_ref[ACC_ROWS:ACC_ROWS + N, 0:1]
    dinv = jnp.where(deg > 0, lax.rsqrt(deg), 0.0)
    h = jnp.dot(x_ref[...], w1_ref[...], preferred_element_type=jnp.float32)
    h1p_ref[...] = h * dinv
    dinv_ref[...] = dinv


def _tc2_body(accp_ref, dinv_ref, b1_ref, g_ref):
    a = accp_ref[0:N, :] + accp_ref[ACC_ROWS:ACC_ROWS + N, :]
    dinv = dinv_ref[...]
    o = jnp.maximum(a * dinv + b1_ref[...], 0.0)
    g_ref[...] = o * dinv


def _tc3_body(accp_ref, dinv_ref, w2_ref, b2_ref, out_ref):
    a = (accp_ref[0:N, :] + accp_ref[ACC_ROWS:ACC_ROWS + N, :]) * dinv_ref[...]
    t = jnp.dot(a, w2_ref[...], preferred_element_type=jnp.float32) + b2_ref[...]
    m = jnp.max(t, axis=1, keepdims=True)
    out_ref[...] = (t - m) - jnp.log(
        jnp.sum(jnp.exp(t - m), axis=1, keepdims=True))


_tc1 = pl.pallas_call(
    _tc1_body,
    out_shape=[jax.ShapeDtypeStruct((N, DH), jnp.float32),
               jax.ShapeDtypeStruct((N, 1), jnp.float32)],
)

_tc2 = pl.pallas_call(
    _tc2_body,
    out_shape=jax.ShapeDtypeStruct((N, DH), jnp.float32),
)

_tc3 = pl.pallas_call(
    _tc3_body,
    out_shape=jax.ShapeDtypeStruct((N, 2), jnp.float32),
)


def kernel(x, edge_index, W1, b1, W2, b2):
    e = edge_index.astype(jnp.int32)
    loops = jnp.arange(N, dtype=jnp.int32)
    src = jnp.concatenate([e[0], loops])
    dst = jnp.concatenate([e[1], loops])
    pad = E_PAD - src.shape[0]
    src = jnp.concatenate([src, jnp.zeros((pad,), jnp.int32)])
    dst = jnp.concatenate([dst, jnp.full((pad,), N, jnp.int32)])
    src3 = src.reshape(NW, CH_PER_TILE, CHUNK)
    dst3 = dst.reshape(NW, CH_PER_TILE, CHUNK)

    zeros = jnp.zeros((ACC_ROWS, DH), jnp.float32)
    ones_blk = jnp.ones((CHUNK, DH), jnp.float32)

    degp = _deg_call(dst3, zeros, ones_blk)
    h1p, dinv = _tc1(x, W1, degp)
    acc1 = _agg_call(h1p, src3, dst3, zeros)
    g = _tc2(acc1, dinv, b1.reshape(1, DH))
    acc2 = _agg_call(g, src3, dst3, zeros)
    return _tc3(acc2, dinv, W2, b2.reshape(1, 2))


# speedup vs baseline: 47.5670x; 1.3982x over previous
import functools

import jax
import jax.numpy as jnp
from jax import lax
from jax.experimental import pallas as pl
from jax.experimental.pallas import tpu as pltpu
from jax.experimental.pallas import tpu_sc as plsc

N = 10000
DH = 16
NC = 2
NS = 16
NW = NC * NS
CHUNK = 128
CH_PER_TILE = 81
E_PAD = NW * CH_PER_TILE * CHUNK
ACC_ROWS = 10112


def _deg_body(dst3, zeros, ones_blk, out, dst_v, ones_v, acc):
    c = lax.axis_index("c")
    s = lax.axis_index("s")
    wid = c * NS + s
    pltpu.sync_copy(dst3.at[wid], dst_v)
    pltpu.sync_copy(ones_blk, ones_v)
    rps = ACC_ROWS // NS
    pltpu.sync_copy(zeros.at[pl.ds(s * rps, rps)], acc.at[pl.ds(s * rps, rps)])
    plsc.subcore_barrier()

    def chunk(j, carry):
        pltpu.sync_copy(ones_v, acc.at[dst_v.at[j]], add=True)
        return carry

    lax.fori_loop(0, CH_PER_TILE, chunk, 0)
    plsc.subcore_barrier()
    pltpu.sync_copy(acc.at[pl.ds(s * rps, rps)],
                    out.at[pl.ds(c * ACC_ROWS + s * rps, rps)])


NBUF = 3


def _agg_body(table, src3, dst3, zeros, out,
              src_v, dst_v, rows_v, acc, sem0, sem1, sem2):
    c = lax.axis_index("c")
    s = lax.axis_index("s")
    wid = c * NS + s
    pltpu.sync_copy(src3.at[wid], src_v)
    pltpu.sync_copy(dst3.at[wid], dst_v)
    rps = ACC_ROWS // NS
    pltpu.sync_copy(zeros.at[pl.ds(s * rps, rps)], acc.at[pl.ds(s * rps, rps)])
    plsc.subcore_barrier()

    sems = (sem0, sem1, sem2)
    for b in range(NBUF):
        pltpu.async_copy(table.at[src_v.at[b]], rows_v.at[b], sems[b])

    def outer(t, carry):
        for b in range(NBUF):
            jj = NBUF * t + b
            pltpu.make_async_copy(table.at[src_v.at[b]], rows_v.at[b],
                                  sems[b]).wait()
            pltpu.sync_copy(rows_v.at[b], acc.at[dst_v.at[jj]], add=True)
            nxt = jj + NBUF

            @pl.when(nxt < CH_PER_TILE)
            def _():
                pltpu.async_copy(table.at[src_v.at[nxt]], rows_v.at[b],
                                 sems[b])
        return carry

    lax.fori_loop(0, CH_PER_TILE // NBUF, outer, 0)
    plsc.subcore_barrier()
    pltpu.sync_copy(acc.at[pl.ds(s * rps, rps)],
                    out.at[pl.ds(c * ACC_ROWS + s * rps, rps)])


_MESH = plsc.VectorSubcoreMesh(core_axis_name="c", subcore_axis_name="s")
_SC_PARAMS = pltpu.CompilerParams(use_tc_tiling_on_sc=False)

_deg_call = pl.kernel(
    _deg_body,
    out_type=jax.ShapeDtypeStruct((NC * ACC_ROWS, DH), jnp.float32),
    mesh=_MESH,
    compiler_params=_SC_PARAMS,
    scratch_types=[
        pltpu.VMEM((CH_PER_TILE, CHUNK), jnp.int32),
        pltpu.VMEM((CHUNK, DH), jnp.float32),
        pltpu.VMEM_SHARED((ACC_ROWS, DH), jnp.float32),
    ],
)

_agg_call = pl.kernel(
    _agg_body,
    out_type=jax.ShapeDtypeStruct((NC * ACC_ROWS, DH), jnp.float32),
    mesh=_MESH,
    compiler_params=_SC_PARAMS,
    scratch_types=[
        pltpu.VMEM((CH_PER_TILE, CHUNK), jnp.int32),
        pltpu.VMEM((CH_PER_TILE, CHUNK), jnp.int32),
        pltpu.VMEM((NBUF, CHUNK, DH), jnp.float32),
        pltpu.VMEM_SHARED((ACC_ROWS, DH), jnp.float32),
        pltpu.SemaphoreType.DMA,
        pltpu.SemaphoreType.DMA,
        pltpu.SemaphoreType.DMA,
    ],
)


def _tc1_body(x_ref, w1_ref, degp_ref, h1p_ref, dinv_ref):
    deg = degp_ref[0:N, 0:1] + degp_ref[ACC_ROWS:ACC_ROWS + N, 0:1]
    dinv = jnp.where(deg > 0, lax.rsqrt(deg), 0.0)
    h = jnp.dot(x_ref[...], w1_ref[...], preferred_element_type=jnp.float32)
    h1p_ref[...] = h * dinv
    dinv_ref[...] = dinv


def _tc2_body(accp_ref, dinv_ref, b1_ref, g_ref):
    a = accp_ref[0:N, :] + accp_ref[ACC_ROWS:ACC_ROWS + N, :]
    dinv = dinv_ref[...]
    o = jnp.maximum(a * dinv + b1_ref[...], 0.0)
    g_ref[...] = o * dinv


def _tc3_body(accp_ref, dinv_ref, w2_ref, b2_ref, out_ref):
    a = (accp_ref[0:N, :] + accp_ref[ACC_ROWS:ACC_ROWS + N, :]) * dinv_ref[...]
    t = jnp.dot(a, w2_ref[...], preferred_element_type=jnp.float32) + b2_ref[...]
    m = jnp.max(t, axis=1, keepdims=True)
    out_ref[...] = (t - m) - jnp.log(
        jnp.sum(jnp.exp(t - m), axis=1, keepdims=True))


_tc1 = pl.pallas_call(
    _tc1_body,
    out_shape=[jax.ShapeDtypeStruct((N, DH), jnp.float32),
               jax.ShapeDtypeStruct((N, 1), jnp.float32)],
)

_tc2 = pl.pallas_call(
    _tc2_body,
    out_shape=jax.ShapeDtypeStruct((N, DH), jnp.float32),
)

_tc3 = pl.pallas_call(
    _tc3_body,
    out_shape=jax.ShapeDtypeStruct((N, 2), jnp.float32),
)


def kernel(x, edge_index, W1, b1, W2, b2):
    e = edge_index.astype(jnp.int32)
    loops = jnp.arange(N, dtype=jnp.int32)
    src = jnp.concatenate([e[0], loops])
    dst = jnp.concatenate([e[1], loops])
    pad = E_PAD - src.shape[0]
    src = jnp.concatenate([src, jnp.zeros((pad,), jnp.int32)])
    dst = jnp.concatenate([dst, jnp.full((pad,), N, jnp.int32)])
    src3 = src.reshape(NW, CH_PER_TILE, CHUNK)
    dst3 = dst.reshape(NW, CH_PER_TILE, CHUNK)

    zeros = jnp.zeros((ACC_ROWS, DH), jnp.float32)
    ones_blk = jnp.ones((CHUNK, DH), jnp.float32)

    degp = _deg_call(dst3, zeros, ones_blk)
    h1p, dinv = _tc1(x, W1, degp)
    acc1 = _agg_call(h1p, src3, dst3, zeros)
    g = _tc2(acc1, dinv, b1.reshape(1, DH))
    acc2 = _agg_call(g, src3, dst3, zeros)
    return _tc3(acc2, dinv, W2, b2.reshape(1, 2))


# speedup vs baseline: 55.1410x; 1.1592x over previous
import functools

import jax
import jax.numpy as jnp
from jax import lax
from jax.experimental import pallas as pl
from jax.experimental.pallas import tpu as pltpu
from jax.experimental.pallas import tpu_sc as plsc

N = 10000
DH = 16
NC = 2
NS = 16
NW = NC * NS
CHUNK = 128
CH_PER_TILE = 81
E_PAD = NW * CH_PER_TILE * CHUNK
ACC_ROWS = 10112


def _deg_body(dst3, zeros, ones_blk, out, dst_v, ones_v, acc):
    c = lax.axis_index("c")
    s = lax.axis_index("s")
    wid = c * NS + s
    pltpu.sync_copy(dst3.at[wid], dst_v)
    pltpu.sync_copy(ones_blk, ones_v)
    rps = ACC_ROWS // NS
    pltpu.sync_copy(zeros.at[pl.ds(s * rps, rps)], acc.at[pl.ds(s * rps, rps)])
    plsc.subcore_barrier()

    def chunk(j, carry):
        pltpu.sync_copy(ones_v, acc.at[dst_v.at[j]], add=True)
        return carry

    lax.fori_loop(0, CH_PER_TILE, chunk, 0)
    plsc.subcore_barrier()
    pltpu.sync_copy(acc.at[pl.ds(s * rps, rps)],
                    out.at[pl.ds(c * ACC_ROWS + s * rps, rps)])


NBUF = 3


def _agg_body(table, src3, dst3, zeros, out,
              src_v, dst_v, rows_v, acc, sem0, sem1, sem2):
    c = lax.axis_index("c")
    s = lax.axis_index("s")
    wid = c * NS + s
    pltpu.sync_copy(src3.at[wid], src_v)
    pltpu.sync_copy(dst3.at[wid], dst_v)
    rps = ACC_ROWS // NS
    pltpu.sync_copy(zeros.at[pl.ds(s * rps, rps)], acc.at[pl.ds(s * rps, rps)])
    plsc.subcore_barrier()

    sems = (sem0, sem1, sem2)
    for b in range(NBUF):
        pltpu.async_copy(table.at[src_v.at[b]], rows_v.at[b], sems[b])

    def outer(t, carry):
        for b in range(NBUF):
            jj = NBUF * t + b
            pltpu.make_async_copy(table.at[src_v.at[b]], rows_v.at[b],
                                  sems[b]).wait()
            pltpu.sync_copy(rows_v.at[b], acc.at[dst_v.at[jj]], add=True)
            nxt = jj + NBUF

            @pl.when(nxt < CH_PER_TILE)
            def _():
                pltpu.async_copy(table.at[src_v.at[nxt]], rows_v.at[b],
                                 sems[b])
        return carry

    lax.fori_loop(0, CH_PER_TILE // NBUF, outer, 0)
    plsc.subcore_barrier()
    pltpu.sync_copy(acc.at[pl.ds(s * rps, rps)],
                    out.at[pl.ds(c * ACC_ROWS + s * rps, rps)])


def _agg2_body(accp, dinv16, b1, src3, dst3, zeros, out,
               src_v, dst_v, rows_v, p0_t, p1_t, d_t, g_t, b1_t,
               acc, gtab, sem0, sem1, sem2):
    c = lax.axis_index("c")
    s = lax.axis_index("s")
    wid = c * NS + s
    rps = ACC_ROWS // NS
    base = s * rps
    pltpu.sync_copy(src3.at[wid], src_v)
    pltpu.sync_copy(dst3.at[wid], dst_v)
    pltpu.sync_copy(accp.at[pl.ds(base, rps)], p0_t)
    pltpu.sync_copy(accp.at[pl.ds(ACC_ROWS + base, rps)], p1_t)
    pltpu.sync_copy(dinv16.at[pl.ds(base, rps)], d_t)
    pltpu.sync_copy(b1, b1_t)
    pltpu.sync_copy(zeros.at[pl.ds(base, rps)], acc.at[pl.ds(base, rps)])

    b1v = b1_t[0]

    def row(i, carry):
        a = p0_t[i] + p1_t[i]
        d = d_t[i]
        g_t[i] = d * jnp.maximum(d * a + b1v, 0.0)
        return carry

    lax.fori_loop(0, rps, row, 0)
    pltpu.sync_copy(g_t, gtab.at[pl.ds(base, rps)])
    plsc.subcore_barrier()

    sems = (sem0, sem1, sem2)
    for b in range(NBUF):
        pltpu.async_copy(gtab.at[src_v.at[b]], rows_v.at[b], sems[b])

    def outer(t, carry):
        for b in range(NBUF):
            jj = NBUF * t + b
            pltpu.make_async_copy(gtab.at[src_v.at[b]], rows_v.at[b],
                                  sems[b]).wait()
            pltpu.sync_copy(rows_v.at[b], acc.at[dst_v.at[jj]], add=True)
            nxt = jj + NBUF

            @pl.when(nxt < CH_PER_TILE)
            def _():
                pltpu.async_copy(gtab.at[src_v.at[nxt]], rows_v.at[b],
                                 sems[b])
        return carry

    lax.fori_loop(0, CH_PER_TILE // NBUF, outer, 0)
    plsc.subcore_barrier()
    pltpu.sync_copy(acc.at[pl.ds(base, rps)],
                    out.at[pl.ds(c * ACC_ROWS + base, rps)])


_MESH = plsc.VectorSubcoreMesh(core_axis_name="c", subcore_axis_name="s")
_SC_PARAMS = pltpu.CompilerParams(use_tc_tiling_on_sc=False)

_deg_call = pl.kernel(
    _deg_body,
    out_type=jax.ShapeDtypeStruct((NC * ACC_ROWS, DH), jnp.float32),
    mesh=_MESH,
    compiler_params=_SC_PARAMS,
    scratch_types=[
        pltpu.VMEM((CH_PER_TILE, CHUNK), jnp.int32),
        pltpu.VMEM((CHUNK, DH), jnp.float32),
        pltpu.VMEM_SHARED((ACC_ROWS, DH), jnp.float32),
    ],
)

_agg_call = pl.kernel(
    _agg_body,
    out_type=jax.ShapeDtypeStruct((NC * ACC_ROWS, DH), jnp.float32),
    mesh=_MESH,
    compiler_params=_SC_PARAMS,
    scratch_types=[
        pltpu.VMEM((CH_PER_TILE, CHUNK), jnp.int32),
        pltpu.VMEM((CH_PER_TILE, CHUNK), jnp.int32),
        pltpu.VMEM((NBUF, CHUNK, DH), jnp.float32),
        pltpu.VMEM_SHARED((ACC_ROWS, DH), jnp.float32),
        pltpu.SemaphoreType.DMA,
        pltpu.SemaphoreType.DMA,
        pltpu.SemaphoreType.DMA,
    ],
)

_RPS = ACC_ROWS // NS

_agg2_call = pl.kernel(
    _agg2_body,
    out_type=jax.ShapeDtypeStruct((NC * ACC_ROWS, DH), jnp.float32),
    mesh=_MESH,
    compiler_params=_SC_PARAMS,
    scratch_types=[
        pltpu.VMEM((CH_PER_TILE, CHUNK), jnp.int32),
        pltpu.VMEM((CH_PER_TILE, CHUNK), jnp.int32),
        pltpu.VMEM((NBUF, CHUNK, DH), jnp.float32),
        pltpu.VMEM((_RPS, DH), jnp.float32),
        pltpu.VMEM((_RPS, DH), jnp.float32),
        pltpu.VMEM((_RPS, DH), jnp.float32),
        pltpu.VMEM((_RPS, DH), jnp.float32),
        pltpu.VMEM((1, DH), jnp.float32),
        pltpu.VMEM_SHARED((ACC_ROWS, DH), jnp.float32),
        pltpu.VMEM_SHARED((ACC_ROWS, DH), jnp.float32),
        pltpu.SemaphoreType.DMA,
        pltpu.SemaphoreType.DMA,
        pltpu.SemaphoreType.DMA,
    ],
)


def _tc1_body(x_ref, w1_ref, degp_ref, h1p_ref, dinv_ref, dinv16_ref):
    deg = degp_ref[0:N, 0:1] + degp_ref[ACC_ROWS:ACC_ROWS + N, 0:1]
    dinv = jnp.where(deg > 0, lax.rsqrt(deg), 0.0)
    h = jnp.dot(x_ref[...], w1_ref[...], preferred_element_type=jnp.float32)
    h1p_ref[...] = h * dinv
    dinv_ref[...] = dinv
    dinv16_ref[0:N, :] = jnp.broadcast_to(dinv, (N, DH))
    dinv16_ref[N:ACC_ROWS, :] = jnp.zeros((ACC_ROWS - N, DH), jnp.float32)


def _tc3_body(accp_ref, dinv_ref, w2_ref, b2_ref, out_ref):
    a = (accp_ref[0:N, :] + accp_ref[ACC_ROWS:ACC_ROWS + N, :]) * dinv_ref[...]
    t = jnp.dot(a, w2_ref[...], preferred_element_type=jnp.float32) + b2_ref[...]
    m = jnp.max(t, axis=1, keepdims=True)
    out_ref[...] = (t - m) - jnp.log(
        jnp.sum(jnp.exp(t - m), axis=1, keepdims=True))


_tc1 = pl.pallas_call(
    _tc1_body,
    out_shape=[jax.ShapeDtypeStruct((N, DH), jnp.float32),
               jax.ShapeDtypeStruct((N, 1), jnp.float32),
               jax.ShapeDtypeStruct((ACC_ROWS, DH), jnp.float32)],
)

_tc3 = pl.pallas_call(
    _tc3_body,
    out_shape=jax.ShapeDtypeStruct((N, 2), jnp.float32),
)


def kernel(x, edge_index, W1, b1, W2, b2):
    e = edge_index.astype(jnp.int32)
    loops = jnp.arange(N, dtype=jnp.int32)
    src = jnp.concatenate([e[0], loops])
    dst = jnp.concatenate([e[1], loops])
    pad = E_PAD - src.shape[0]
    src = jnp.concatenate([src, jnp.zeros((pad,), jnp.int32)])
    dst = jnp.concatenate([dst, jnp.full((pad,), N, jnp.int32)])
    src3 = src.reshape(NW, CH_PER_TILE, CHUNK)
    dst3 = dst.reshape(NW, CH_PER_TILE, CHUNK)

    zeros = jnp.zeros((ACC_ROWS, DH), jnp.float32)
    ones_blk = jnp.ones((CHUNK, DH), jnp.float32)

    degp = _deg_call(dst3, zeros, ones_blk)
    h1p, dinv, dinv16 = _tc1(x, W1, degp)
    acc1 = _agg_call(h1p, src3, dst3, zeros)
    acc2 = _agg2_call(acc1, dinv16, b1.reshape(1, DH), src3, dst3, zeros)
    return _tc3(acc2, dinv, W2, b2.reshape(1, 2))


# speedup vs baseline: 59.8808x; 1.0860x over previous
import functools

import jax
import jax.numpy as jnp
from jax import lax
from jax.experimental import pallas as pl
from jax.experimental.pallas import tpu as pltpu
from jax.experimental.pallas import tpu_sc as plsc

N = 10000
DH = 16
NC = 2
NS = 16
NW = NC * NS
CHUNK = 128
CH_PER_TILE = 81
E_PAD = NW * CH_PER_TILE * CHUNK
ACC_ROWS = 10112


def _deg_body(dst3, zeros, ones_blk, out, dst_v, ones_v, acc):
    c = lax.axis_index("c")
    s = lax.axis_index("s")
    wid = c * NS + s
    pltpu.sync_copy(dst3.at[wid], dst_v)
    pltpu.sync_copy(ones_blk, ones_v)
    rps = ACC_ROWS // NS
    pltpu.sync_copy(zeros.at[pl.ds(s * rps, rps)], acc.at[pl.ds(s * rps, rps)])
    plsc.subcore_barrier()

    def chunk(j, carry):
        pltpu.sync_copy(ones_v, acc.at[dst_v.at[j]], add=True)
        return carry

    lax.fori_loop(0, CH_PER_TILE, chunk, 0)
    plsc.subcore_barrier()
    pltpu.sync_copy(acc.at[pl.ds(s * rps, rps)],
                    out.at[pl.ds(c * ACC_ROWS + s * rps, rps)])


NBUF = 3


def _agg_body(table, src3, dst3, zeros, out,
              src_v, dst_v, rows_v, acc, gtab, sem0, sem1, sem2):
    c = lax.axis_index("c")
    s = lax.axis_index("s")
    wid = c * NS + s
    rps = ACC_ROWS // NS
    base = s * rps
    pltpu.sync_copy(src3.at[wid], src_v)
    pltpu.sync_copy(dst3.at[wid], dst_v)
    pltpu.sync_copy(table.at[pl.ds(base, rps)], gtab.at[pl.ds(base, rps)])
    pltpu.sync_copy(zeros.at[pl.ds(base, rps)], acc.at[pl.ds(base, rps)])
    plsc.subcore_barrier()

    sems = (sem0, sem1, sem2)
    for b in range(NBUF):
        pltpu.async_copy(gtab.at[src_v.at[b]], rows_v.at[b], sems[b])

    def outer(t, carry):
        for b in range(NBUF):
            jj = NBUF * t + b
            pltpu.make_async_copy(gtab.at[src_v.at[b]], rows_v.at[b],
                                  sems[b]).wait()
            pltpu.sync_copy(rows_v.at[b], acc.at[dst_v.at[jj]], add=True)
            nxt = jj + NBUF

            @pl.when(nxt < CH_PER_TILE)
            def _():
                pltpu.async_copy(gtab.at[src_v.at[nxt]], rows_v.at[b],
                                 sems[b])
        return carry

    lax.fori_loop(0, CH_PER_TILE // NBUF, outer, 0)
    plsc.subcore_barrier()
    pltpu.sync_copy(acc.at[pl.ds(base, rps)],
                    out.at[pl.ds(c * ACC_ROWS + base, rps)])


def _agg2_body(accp, dinv16, b1, src3, dst3, zeros, out,
               src_v, dst_v, rows_v, p0_t, p1_t, d_t, g_t, b1_t,
               acc, gtab, sem0, sem1, sem2):
    c = lax.axis_index("c")
    s = lax.axis_index("s")
    wid = c * NS + s
    rps = ACC_ROWS // NS
    base = s * rps
    pltpu.sync_copy(src3.at[wid], src_v)
    pltpu.sync_copy(dst3.at[wid], dst_v)
    pltpu.sync_copy(accp.at[pl.ds(base, rps)], p0_t)
    pltpu.sync_copy(accp.at[pl.ds(ACC_ROWS + base, rps)], p1_t)
    pltpu.sync_copy(dinv16.at[pl.ds(base, rps)], d_t)
    pltpu.sync_copy(b1, b1_t)
    pltpu.sync_copy(zeros.at[pl.ds(base, rps)], acc.at[pl.ds(base, rps)])

    b1v = b1_t[0]

    def row(i, carry):
        a = p0_t[i] + p1_t[i]
        d = d_t[i]
        g_t[i] = d * jnp.maximum(d * a + b1v, 0.0)
        return carry

    lax.fori_loop(0, rps, row, 0)
    pltpu.sync_copy(g_t, gtab.at[pl.ds(base, rps)])
    plsc.subcore_barrier()

    sems = (sem0, sem1, sem2)
    for b in range(NBUF):
        pltpu.async_copy(gtab.at[src_v.at[b]], rows_v.at[b], sems[b])

    def outer(t, carry):
        for b in range(NBUF):
            jj = NBUF * t + b
            pltpu.make_async_copy(gtab.at[src_v.at[b]], rows_v.at[b],
                                  sems[b]).wait()
            pltpu.sync_copy(rows_v.at[b], acc.at[dst_v.at[jj]], add=True)
            nxt = jj + NBUF

            @pl.when(nxt < CH_PER_TILE)
            def _():
                pltpu.async_copy(gtab.at[src_v.at[nxt]], rows_v.at[b],
                                 sems[b])
        return carry

    lax.fori_loop(0, CH_PER_TILE // NBUF, outer, 0)
    plsc.subcore_barrier()
    pltpu.sync_copy(acc.at[pl.ds(base, rps)],
                    out.at[pl.ds(c * ACC_ROWS + base, rps)])


_MESH = plsc.VectorSubcoreMesh(core_axis_name="c", subcore_axis_name="s")
_SC_PARAMS = pltpu.CompilerParams(use_tc_tiling_on_sc=False)

_deg_call = pl.kernel(
    _deg_body,
    out_type=jax.ShapeDtypeStruct((NC * ACC_ROWS, DH), jnp.float32),
    mesh=_MESH,
    compiler_params=_SC_PARAMS,
    scratch_types=[
        pltpu.VMEM((CH_PER_TILE, CHUNK), jnp.int32),
        pltpu.VMEM((CHUNK, DH), jnp.float32),
        pltpu.VMEM_SHARED((ACC_ROWS, DH), jnp.float32),
    ],
)

_agg_call = pl.kernel(
    _agg_body,
    out_type=jax.ShapeDtypeStruct((NC * ACC_ROWS, DH), jnp.float32),
    mesh=_MESH,
    compiler_params=_SC_PARAMS,
    scratch_types=[
        pltpu.VMEM((CH_PER_TILE, CHUNK), jnp.int32),
        pltpu.VMEM((CH_PER_TILE, CHUNK), jnp.int32),
        pltpu.VMEM((NBUF, CHUNK, DH), jnp.float32),
        pltpu.VMEM_SHARED((ACC_ROWS, DH), jnp.float32),
        pltpu.VMEM_SHARED((ACC_ROWS, DH), jnp.float32),
        pltpu.SemaphoreType.DMA,
        pltpu.SemaphoreType.DMA,
        pltpu.SemaphoreType.DMA,
    ],
)

_RPS = ACC_ROWS // NS

_agg2_call = pl.kernel(
    _agg2_body,
    out_type=jax.ShapeDtypeStruct((NC * ACC_ROWS, DH), jnp.float32),
    mesh=_MESH,
    compiler_params=_SC_PARAMS,
    scratch_types=[
        pltpu.VMEM((CH_PER_TILE, CHUNK), jnp.int32),
        pltpu.VMEM((CH_PER_TILE, CHUNK), jnp.int32),
        pltpu.VMEM((NBUF, CHUNK, DH), jnp.float32),
        pltpu.VMEM((_RPS, DH), jnp.float32),
        pltpu.VMEM((_RPS, DH), jnp.float32),
        pltpu.VMEM((_RPS, DH), jnp.float32),
        pltpu.VMEM((_RPS, DH), jnp.float32),
        pltpu.VMEM((1, DH), jnp.float32),
        pltpu.VMEM_SHARED((ACC_ROWS, DH), jnp.float32),
        pltpu.VMEM_SHARED((ACC_ROWS, DH), jnp.float32),
        pltpu.SemaphoreType.DMA,
        pltpu.SemaphoreType.DMA,
        pltpu.SemaphoreType.DMA,
    ],
)


def _tc1_body(x_ref, w1_ref, degp_ref, h1p_ref, dinv_ref, dinv16_ref):
    deg = degp_ref[0:N, 0:1] + degp_ref[ACC_ROWS:ACC_ROWS + N, 0:1]
    dinv = jnp.where(deg > 0, lax.rsqrt(deg), 0.0)
    h = jnp.dot(x_ref[...], w1_ref[...], preferred_element_type=jnp.float32)
    h1p_ref[0:N, :] = h * dinv
    h1p_ref[N:ACC_ROWS, :] = jnp.zeros((ACC_ROWS - N, DH), jnp.float32)
    dinv_ref[...] = dinv
    dinv16_ref[0:N, :] = jnp.broadcast_to(dinv, (N, DH))
    dinv16_ref[N:ACC_ROWS, :] = jnp.zeros((ACC_ROWS - N, DH), jnp.float32)


def _tc3_body(accp_ref, dinv_ref, w2_ref, b2_ref, out_ref):
    a = (accp_ref[0:N, :] + accp_ref[ACC_ROWS:ACC_ROWS + N, :]) * dinv_ref[...]
    t = jnp.dot(a, w2_ref[...], preferred_element_type=jnp.float32) + b2_ref[...]
    m = jnp.max(t, axis=1, keepdims=True)
    out_ref[...] = (t - m) - jnp.log(
        jnp.sum(jnp.exp(t - m), axis=1, keepdims=True))


_tc1 = pl.pallas_call(
    _tc1_body,
    out_shape=[jax.ShapeDtypeStruct((ACC_ROWS, DH), jnp.float32),
               jax.ShapeDtypeStruct((N, 1), jnp.float32),
               jax.ShapeDtypeStruct((ACC_ROWS, DH), jnp.float32)],
)

_tc3 = pl.pallas_call(
    _tc3_body,
    out_shape=jax.ShapeDtypeStruct((N, 2), jnp.float32),
)


def kernel(x, edge_index, W1, b1, W2, b2):
    e = edge_index.astype(jnp.int32)
    loops = jnp.arange(N, dtype=jnp.int32)
    src = jnp.concatenate([e[0], loops])
    dst = jnp.concatenate([e[1], loops])
    pad = E_PAD - src.shape[0]
    src = jnp.concatenate([src, jnp.zeros((pad,), jnp.int32)])
    dst = jnp.concatenate([dst, jnp.full((pad,), N, jnp.int32)])
    src3 = src.reshape(NW, CH_PER_TILE, CHUNK)
    dst3 = dst.reshape(NW, CH_PER_TILE, CHUNK)

    zeros = jnp.zeros((ACC_ROWS, DH), jnp.float32)
    ones_blk = jnp.ones((CHUNK, DH), jnp.float32)

    degp = _deg_call(dst3, zeros, ones_blk)
    h1p, dinv, dinv16 = _tc1(x, W1, degp)
    acc1 = _agg_call(h1p, src3, dst3, zeros)
    acc2 = _agg2_call(acc1, dinv16, b1.reshape(1, DH), src3, dst3, zeros)
    return _tc3(acc2, dinv, W2, b2.reshape(1, 2))


# speedup vs baseline: 61.0385x; 1.0193x over previous
import functools

import jax
import jax.numpy as jnp
from jax import lax
from jax.experimental import pallas as pl
from jax.experimental.pallas import tpu as pltpu
from jax.experimental.pallas import tpu_sc as plsc

N = 10000
DH = 16
NC = 2
NS = 16
NW = NC * NS
CHUNK = 128
CH_PER_TILE = 81
E_PAD = NW * CH_PER_TILE * CHUNK
ACC_ROWS = 10112


DEG_W = 8


def _deg_body(dst3, zeros, ones_blk, out, dst_v, ones_v, acc, sem):
    c = lax.axis_index("c")
    s = lax.axis_index("s")
    wid = c * NS + s
    pltpu.sync_copy(dst3.at[wid], dst_v)
    pltpu.sync_copy(ones_blk, ones_v)
    rps = ACC_ROWS // NS
    pltpu.sync_copy(zeros.at[pl.ds(s * rps, rps)], acc.at[pl.ds(s * rps, rps)])
    plsc.subcore_barrier()

    def chunk(j, carry):
        pltpu.async_copy(ones_v, acc.at[dst_v.at[j]], sem, add=True)

        @pl.when(j >= DEG_W)
        def _():
            pltpu.make_async_copy(ones_v, acc.at[dst_v.at[j]], sem).wait()

        return carry

    lax.fori_loop(0, CH_PER_TILE, chunk, 0)

    def drain(j, carry):
        pltpu.make_async_copy(ones_v, acc.at[dst_v.at[j]], sem).wait()
        return carry

    lax.fori_loop(0, DEG_W, drain, 0)
    plsc.subcore_barrier()
    pltpu.sync_copy(acc.at[pl.ds(s * rps, rps)],
                    out.at[pl.ds(c * ACC_ROWS + s * rps, rps)])


NBUF = 3


def _agg_body(table, src3, dst3, zeros, out,
              src_v, dst_v, rows_v, acc, gtab, sem0, sem1, sem2):
    c = lax.axis_index("c")
    s = lax.axis_index("s")
    wid = c * NS + s
    rps = ACC_ROWS // NS
    base = s * rps
    pltpu.sync_copy(src3.at[wid], src_v)
    pltpu.sync_copy(dst3.at[wid], dst_v)
    pltpu.sync_copy(table.at[pl.ds(base, rps)], gtab.at[pl.ds(base, rps)])
    pltpu.sync_copy(zeros.at[pl.ds(base, rps)], acc.at[pl.ds(base, rps)])
    plsc.subcore_barrier()

    sems = (sem0, sem1, sem2)
    for b in range(NBUF):
        pltpu.async_copy(gtab.at[src_v.at[b]], rows_v.at[b], sems[b])

    def outer(t, carry):
        for b in range(NBUF):
            jj = NBUF * t + b
            pltpu.make_async_copy(gtab.at[src_v.at[b]], rows_v.at[b],
                                  sems[b]).wait()
            pltpu.sync_copy(rows_v.at[b], acc.at[dst_v.at[jj]], add=True)
            nxt = jj + NBUF

            @pl.when(nxt < CH_PER_TILE)
            def _():
                pltpu.async_copy(gtab.at[src_v.at[nxt]], rows_v.at[b],
                                 sems[b])
        return carry

    lax.fori_loop(0, CH_PER_TILE // NBUF, outer, 0)
    plsc.subcore_barrier()
    pltpu.sync_copy(acc.at[pl.ds(base, rps)],
                    out.at[pl.ds(c * ACC_ROWS + base, rps)])


def _agg2_body(accp, dinv16, b1, src3, dst3, zeros, out,
               src_v, dst_v, rows_v, p0_t, p1_t, d_t, g_t, b1_t,
               acc, gtab, sem0, sem1, sem2):
    c = lax.axis_index("c")
    s = lax.axis_index("s")
    wid = c * NS + s
    rps = ACC_ROWS // NS
    base = s * rps
    pltpu.sync_copy(src3.at[wid], src_v)
    pltpu.sync_copy(dst3.at[wid], dst_v)
    pltpu.sync_copy(accp.at[pl.ds(base, rps)], p0_t)
    pltpu.sync_copy(accp.at[pl.ds(ACC_ROWS + base, rps)], p1_t)
    pltpu.sync_copy(dinv16.at[pl.ds(base, rps)], d_t)
    pltpu.sync_copy(b1, b1_t)
    pltpu.sync_copy(zeros.at[pl.ds(base, rps)], acc.at[pl.ds(base, rps)])

    b1v = b1_t[0]

    def row(i, carry):
        a = p0_t[i] + p1_t[i]
        d = d_t[i]
        g_t[i] = d * jnp.maximum(d * a + b1v, 0.0)
        return carry

    lax.fori_loop(0, rps, row, 0)
    pltpu.sync_copy(g_t, gtab.at[pl.ds(base, rps)])
    plsc.subcore_barrier()

    sems = (sem0, sem1, sem2)
    for b in range(NBUF):
        pltpu.async_copy(gtab.at[src_v.at[b]], rows_v.at[b], sems[b])

    def outer(t, carry):
        for b in range(NBUF):
            jj = NBUF * t + b
            pltpu.make_async_copy(gtab.at[src_v.at[b]], rows_v.at[b],
                                  sems[b]).wait()
            pltpu.sync_copy(rows_v.at[b], acc.at[dst_v.at[jj]], add=True)
            nxt = jj + NBUF

            @pl.when(nxt < CH_PER_TILE)
            def _():
                pltpu.async_copy(gtab.at[src_v.at[nxt]], rows_v.at[b],
                                 sems[b])
        return carry

    lax.fori_loop(0, CH_PER_TILE // NBUF, outer, 0)
    plsc.subcore_barrier()
    pltpu.sync_copy(acc.at[pl.ds(base, rps)],
                    out.at[pl.ds(c * ACC_ROWS + base, rps)])


_MESH = plsc.VectorSubcoreMesh(core_axis_name="c", subcore_axis_name="s")
_SC_PARAMS = pltpu.CompilerParams(use_tc_tiling_on_sc=False)

_deg_call = pl.kernel(
    _deg_body,
    out_type=jax.ShapeDtypeStruct((NC * ACC_ROWS, DH), jnp.float32),
    mesh=_MESH,
    compiler_params=_SC_PARAMS,
    scratch_types=[
        pltpu.VMEM((CH_PER_TILE, CHUNK), jnp.int32),
        pltpu.VMEM((CHUNK, DH), jnp.float32),
        pltpu.VMEM_SHARED((ACC_ROWS, DH), jnp.float32),
        pltpu.SemaphoreType.DMA,
    ],
)

_agg_call = pl.kernel(
    _agg_body,
    out_type=jax.ShapeDtypeStruct((NC * ACC_ROWS, DH), jnp.float32),
    mesh=_MESH,
    compiler_params=_SC_PARAMS,
    scratch_types=[
        pltpu.VMEM((CH_PER_TILE, CHUNK), jnp.int32),
        pltpu.VMEM((CH_PER_TILE, CHUNK), jnp.int32),
        pltpu.VMEM((NBUF, CHUNK, DH), jnp.float32),
        pltpu.VMEM_SHARED((ACC_ROWS, DH), jnp.float32),
        pltpu.VMEM_SHARED((ACC_ROWS, DH), jnp.float32),
        pltpu.SemaphoreType.DMA,
        pltpu.SemaphoreType.DMA,
        pltpu.SemaphoreType.DMA,
    ],
)

_RPS = ACC_ROWS // NS

_agg2_call = pl.kernel(
    _agg2_body,
    out_type=jax.ShapeDtypeStruct((NC * ACC_ROWS, DH), jnp.float32),
    mesh=_MESH,
    compiler_params=_SC_PARAMS,
    scratch_types=[
        pltpu.VMEM((CH_PER_TILE, CHUNK), jnp.int32),
        pltpu.VMEM((CH_PER_TILE, CHUNK), jnp.int32),
        pltpu.VMEM((NBUF, CHUNK, DH), jnp.float32),
        pltpu.VMEM((_RPS, DH), jnp.float32),
        pltpu.VMEM((_RPS, DH), jnp.float32),
        pltpu.VMEM((_RPS, DH), jnp.float32),
        pltpu.VMEM((_RPS, DH), jnp.float32),
        pltpu.VMEM((1, DH), jnp.float32),
        pltpu.VMEM_SHARED((ACC_ROWS, DH), jnp.float32),
        pltpu.VMEM_SHARED((ACC_ROWS, DH), jnp.float32),
        pltpu.SemaphoreType.DMA,
        pltpu.SemaphoreType.DMA,
        pltpu.SemaphoreType.DMA,
    ],
)


def _tc1_body(x_ref, w1_ref, degp_ref, h1p_ref, dinv_ref, dinv16_ref):
    deg = degp_ref[0:N, 0:1] + degp_ref[ACC_ROWS:ACC_ROWS + N, 0:1]
    dinv = jnp.where(deg > 0, lax.rsqrt(deg), 0.0)
    h = jnp.dot(x_ref[...], w1_ref[...], preferred_element_type=jnp.float32)
    h1p_ref[0:N, :] = h * dinv
    h1p_ref[N:ACC_ROWS, :] = jnp.zeros((ACC_ROWS - N, DH), jnp.float32)
    dinv_ref[...] = dinv
    dinv16_ref[0:N, :] = jnp.broadcast_to(dinv, (N, DH))
    dinv16_ref[N:ACC_ROWS, :] = jnp.zeros((ACC_ROWS - N, DH), jnp.float32)


def _tc3_body(accp_ref, dinv_ref, w2_ref, b2_ref, out_ref):
    a = (accp_ref[0:N, :] + accp_ref[ACC_ROWS:ACC_ROWS + N, :]) * dinv_ref[...]
    t = jnp.dot(a, w2_ref[...], preferred_element_type=jnp.float32) + b2_ref[...]
    m = jnp.max(t, axis=1, keepdims=True)
    out_ref[...] = (t - m) - jnp.log(
        jnp.sum(jnp.exp(t - m), axis=1, keepdims=True))


_tc1 = pl.pallas_call(
    _tc1_body,
    out_shape=[jax.ShapeDtypeStruct((ACC_ROWS, DH), jnp.float32),
               jax.ShapeDtypeStruct((N, 1), jnp.float32),
               jax.ShapeDtypeStruct((ACC_ROWS, DH), jnp.float32)],
)

_tc3 = pl.pallas_call(
    _tc3_body,
    out_shape=jax.ShapeDtypeStruct((N, 2), jnp.float32),
)


def kernel(x, edge_index, W1, b1, W2, b2):
    e = edge_index.astype(jnp.int32)
    loops = jnp.arange(N, dtype=jnp.int32)
    src = jnp.concatenate([e[0], loops])
    dst = jnp.concatenate([e[1], loops])
    pad = E_PAD - src.shape[0]
    src = jnp.concatenate([src, jnp.zeros((pad,), jnp.int32)])
    dst = jnp.concatenate([dst, jnp.full((pad,), N, jnp.int32)])
    src3 = src.reshape(NW, CH_PER_TILE, CHUNK)
    dst3 = dst.reshape(NW, CH_PER_TILE, CHUNK)

    zeros = jnp.zeros((ACC_ROWS, DH), jnp.float32)
    ones_blk = jnp.ones((CHUNK, DH), jnp.float32)

    degp = _deg_call(dst3, zeros, ones_blk)
    h1p, dinv, dinv16 = _tc1(x, W1, degp)
    acc1 = _agg_call(h1p, src3, dst3, zeros)
    acc2 = _agg2_call(acc1, dinv16, b1.reshape(1, DH), src3, dst3, zeros)
    return _tc3(acc2, dinv, W2, b2.reshape(1, 2))
